# Initial kernel scaffold; baseline (speedup 1.0000x reference)
import jax
import jax.numpy as jnp
from jax.experimental import pallas as pl


def kernel(x, edge_index, edge_attr, e_w0, e_b0, e_w1, e_b1, e_w2, e_b2, e_w3, e_b3, e_w4, e_b4, e_w5, e_b5, n_w0, n_b0, n_w1, n_b1, n_w2, n_b2):
    raise NotImplementedError("write your pallas kernel here")


# speedup vs baseline: 2.8557x; 2.8557x over previous
import functools

import jax
import jax.numpy as jnp
from jax import lax
from jax.experimental import pallas as pl
from jax.experimental.pallas import tpu as pltpu
from jax.experimental.pallas import tpu_sc as plsc

F32 = jnp.float32

_N = 10000
_E = 320000
_NW = 32
_EPT = _E // _NW
_CH = 80
_NCH = _EPT // _CH
_ROWS_PER_TILE = _N // 16
_D = 16
_MD = 64
_BLK = 3200


def _sc_mesh():
    return plsc.VectorSubcoreMesh(core_axis_name="c", subcore_axis_name="s")


def _gather_body(ridx_hbm, sidx_hbm, xpad_hbm, xr_hbm, xs_hbm,
                 ridx_v, sidx_v, bufr, bufs, sem):
    c = lax.axis_index("c")
    s = lax.axis_index("s")
    wid = s * 2 + c
    pltpu.sync_copy(ridx_hbm.at[wid], ridx_v)
    pltpu.sync_copy(sidx_hbm.at[wid], sidx_v)
    base = wid * _EPT

    def body(j, carry):
        cr = pltpu.async_copy(xpad_hbm.at[ridx_v.at[j]], bufr, sem)
        cs = pltpu.async_copy(xpad_hbm.at[sidx_v.at[j]], bufs, sem)
        cr.wait()
        cs.wait()
        row0 = base + j * _CH
        pltpu.sync_copy(bufr, xr_hbm.at[pl.ds(row0, _CH)])
        pltpu.sync_copy(bufs, xs_hbm.at[pl.ds(row0, _CH)])
        return carry

    lax.fori_loop(0, _NCH, body, 0)


def _sc_gather(ridx3, sidx3, xpad):
    run = functools.partial(
        pl.kernel,
        out_type=(jax.ShapeDtypeStruct((_E, _D), F32),
                  jax.ShapeDtypeStruct((_E, _D), F32)),
        mesh=_sc_mesh(),
        scratch_types=[
            pltpu.VMEM((_NCH, _CH), jnp.int32),
            pltpu.VMEM((_NCH, _CH), jnp.int32),
            pltpu.VMEM((_CH, _D), F32),
            pltpu.VMEM((_CH, _D), F32),
            pltpu.SemaphoreType.DMA,
        ],
        compiler_params=pltpu.CompilerParams(use_tc_tiling_on_sc=False),
    )(_gather_body)
    return run(ridx3, sidx3, xpad)


def _scatter_body(ridx_hbm, msg_hbm, zeros_hbm, out_hbm,
                  ridx_v, mbuf, zbuf, acc, sem):
    c = lax.axis_index("c")
    s = lax.axis_index("s")
    wid = s * 2 + c
    pltpu.sync_copy(zeros_hbm, zbuf)
    pltpu.sync_copy(zbuf, acc.at[pl.ds(s * _ROWS_PER_TILE, _ROWS_PER_TILE)])
    pltpu.sync_copy(ridx_hbm.at[wid], ridx_v)
    plsc.subcore_barrier()
    base = wid * _EPT

    def body(j, carry):
        pltpu.sync_copy(msg_hbm.at[pl.ds(base + j * _CH, _CH)], mbuf)
        pltpu.sync_copy(mbuf, acc.at[ridx_v.at[j]], add=True)
        return carry

    lax.fori_loop(0, _NCH, body, 0)
    plsc.subcore_barrier()
    r0 = s * _ROWS_PER_TILE
    pltpu.sync_copy(acc.at[pl.ds(r0, _ROWS_PER_TILE)], zbuf)
    pltpu.sync_copy(zbuf, out_hbm.at[c, pl.ds(r0, _ROWS_PER_TILE)])


def _sc_scatter(ridx3, msg, zeros):
    run = functools.partial(
        pl.kernel,
        out_type=jax.ShapeDtypeStruct((2, _N, _MD), F32),
        mesh=_sc_mesh(),
        scratch_types=[
            pltpu.VMEM((_NCH, _CH), jnp.int32),
            pltpu.VMEM((_CH, _MD), F32),
            pltpu.VMEM((_ROWS_PER_TILE, _MD), F32),
            pltpu.VMEM_SHARED((_N, _MD), F32),
            pltpu.SemaphoreType.DMA,
        ],
        compiler_params=pltpu.CompilerParams(use_tc_tiling_on_sc=False),
    )(_scatter_body)
    return run(ridx3, msg, zeros)


def _edge_mlp_body(xr_ref, xs_ref, w0r_ref, w0s_ref, b0_ref,
                   w1_ref, b1_ref, w2_ref, b2_ref, w3_ref, b3_ref,
                   w4_ref, b4_ref, w5_ref, b5_ref, out_ref):
    dot = functools.partial(jnp.dot, preferred_element_type=F32)
    h = dot(xr_ref[...], w0r_ref[...]) + dot(xs_ref[...], w0s_ref[...])
    h = jnp.maximum(h + b0_ref[...], 0.0)
    h = jnp.maximum(dot(h, w1_ref[...]) + b1_ref[...], 0.0)
    h = jnp.maximum(dot(h, w2_ref[...]) + b2_ref[...], 0.0)
    h = jnp.maximum(dot(h, w3_ref[...]) + b3_ref[...], 0.0)
    h = jnp.maximum(dot(h, w4_ref[...]) + b4_ref[...], 0.0)
    out_ref[...] = dot(h, w5_ref[...]) + b5_ref[...]


def _edge_mlp(xr, xs, w0r, w0s, b0, w1, b1, w2, b2, w3, b3, w4, b4, w5, b5):
    nblk = _E // _BLK
    full = lambda shape: pl.BlockSpec(shape, lambda i: (0, 0))
    return pl.pallas_call(
        _edge_mlp_body,
        grid=(nblk,),
        in_specs=[
            pl.BlockSpec((_BLK, _D), lambda i: (i, 0)),
            pl.BlockSpec((_BLK, _D), lambda i: (i, 0)),
            full(w0r.shape), full(w0s.shape), full(b0.shape),
            full(w1.shape), full(b1.shape), full(w2.shape), full(b2.shape),
            full(w3.shape), full(b3.shape), full(w4.shape), full(b4.shape),
            full(w5.shape), full(b5.shape),
        ],
        out_specs=pl.BlockSpec((_BLK, _MD), lambda i: (i, 0)),
        out_shape=jax.ShapeDtypeStruct((_E, _MD), F32),
        compiler_params=pltpu.CompilerParams(
            dimension_semantics=("arbitrary",)),
    )(xr, xs, w0r, w0s, b0, w1, b1, w2, b2, w3, b3, w4, b4, w5, b5)


def _node_mlp_body(x_ref, p0_ref, p1_ref, w0x_ref, w0a_ref, b0_ref,
                   w1_ref, b1_ref, w2_ref, b2_ref, out_ref):
    dot = functools.partial(jnp.dot, preferred_element_type=F32)
    aggr = p0_ref[...] + p1_ref[...]
    h = dot(x_ref[...], w0x_ref[...]) + dot(aggr, w0a_ref[...])
    h = jnp.maximum(h + b0_ref[...], 0.0)
    h = jnp.maximum(dot(h, w1_ref[...]) + b1_ref[...], 0.0)
    out_ref[...] = dot(h, w2_ref[...]) + b2_ref[...]


def _node_mlp(x, p0, p1, w0x, w0a, b0, w1, b1, w2, b2):
    return pl.pallas_call(
        _node_mlp_body,
        out_shape=jax.ShapeDtypeStruct((_N, 2), F32),
    )(x, p0, p1, w0x, w0a, b0, w1, b1, w2, b2)


def kernel(x, edge_index, edge_attr,
           e_w0, e_b0, e_w1, e_b1, e_w2, e_b2, e_w3, e_b3, e_w4, e_b4,
           e_w5, e_b5, n_w0, n_b0, n_w1, n_b1, n_w2, n_b2):
    del edge_attr
    sender = edge_index[0]
    receiver = edge_index[1]
    ridx3 = receiver.reshape(_NW, _NCH, _CH)
    sidx3 = sender.reshape(_NW, _NCH, _CH)
    xpad = jnp.zeros((_N, _D), F32).at[:, :5].set(x)

    a, b, cmat = e_w0[:, 0:5], e_w0[:, 5:10], e_w0[:, 10:15]
    w0r = jnp.zeros((_D, 150), F32).at[0:5, :].set((a + cmat).T)
    w0s = jnp.zeros((_D, 150), F32).at[0:5, :].set((b - cmat).T)
    w5 = jnp.zeros((150, _MD), F32).at[:, 0:50].set(e_w5.T)
    b5 = jnp.zeros((1, _MD), F32).at[:, 0:50].set(e_b5)

    xr, xs = _sc_gather(ridx3, sidx3, xpad)
    msg = _edge_mlp(xr, xs, w0r, w0s, e_b0.reshape(1, -1),
                    e_w1.T, e_b1.reshape(1, -1), e_w2.T, e_b2.reshape(1, -1),
                    e_w3.T, e_b3.reshape(1, -1), e_w4.T, e_b4.reshape(1, -1),
                    w5, b5)

    zeros = jnp.zeros((_ROWS_PER_TILE, _MD), F32)
    partials = _sc_scatter(ridx3, msg, zeros)

    w0x = n_w0[:, 0:5].T
    w0a = jnp.zeros((_MD, 100), F32).at[0:50, :].set(n_w0[:, 5:55].T)
    out = _node_mlp(x, partials[0], partials[1], w0x, w0a,
                    n_b0.reshape(1, -1), n_w1.T, n_b1.reshape(1, -1),
                    n_w2.T, n_b2.reshape(1, -1))
    return out
